# baseline (speedup 1.0000x reference)
import functools

import jax
import jax.numpy as jnp
from jax import lax
from jax.experimental import pallas as pl
from jax.experimental.pallas import tpu as pltpu, tpu_sc as plsc

N = 10000
E = 320000
DF = 128
H = 30
HP = 32
KCH = 5
NW = 32
ESH = E // NW
CHUNK = 4000
NBUF = 4
NCHUNK = E // CHUNK
GROUPS = CHUNK // 16
NG = N // 16

_mesh = plsc.VectorSubcoreMesh(core_axis_name="c", subcore_axis_name="s")
_sc_params = pltpu.CompilerParams(needs_layout_passes=False)


def _wid():
    return lax.axis_index("s") * 2 + lax.axis_index("c")


@functools.partial(
    pl.kernel,
    out_type=(jax.ShapeDtypeStruct((NW * N,), jnp.float32),
              jax.ShapeDtypeStruct((E,), jnp.int32)),
    mesh=_mesh,
    scratch_types=[pltpu.VMEM((ESH,), jnp.int32),
                   pltpu.VMEM((ESH,), jnp.int32),
                   pltpu.VMEM((ESH,), jnp.int32),
                   pltpu.VMEM((N,), jnp.float32)],
    compiler_params=_sc_params,
)
def _deg_pack(edge_hbm, degp_hbm, packed_hbm, row_v, col_v, pk_v, deg_v):
    w = _wid()
    base = w * ESH
    pltpu.sync_copy(edge_hbm.at[pl.ds(base, ESH)], row_v)
    pltpu.sync_copy(edge_hbm.at[pl.ds(E + base, ESH)], col_v)
    zero = jnp.zeros((16,), jnp.float32)

    @plsc.parallel_loop(0, N, step=16, unroll=10)
    def _(i):
        deg_v[pl.ds(i, 16)] = zero

    ones = jnp.ones((16,), jnp.float32)
    k14 = jnp.full((16,), 16384, jnp.int32)

    @plsc.parallel_loop(0, ESH, step=16, unroll=10)
    def _(g):
        s = pl.ds(g, 16)
        r = row_v[s]
        pk_v[s] = r * k14 + col_v[s]
        plsc.addupdate_scatter(deg_v, [r], ones)

    pltpu.sync_copy(pk_v, packed_hbm.at[pl.ds(base, ESH)])
    pltpu.sync_copy(deg_v, degp_hbm.at[pl.ds(w * N, N)])


@functools.partial(
    pl.kernel,
    out_type=jax.ShapeDtypeStruct((E,), jnp.float32),
    mesh=_mesh,
    scratch_types=[pltpu.VMEM((N,), jnp.float32),
                   pltpu.VMEM((ESH,), jnp.int32),
                   pltpu.VMEM((ESH,), jnp.float32)],
    compiler_params=_sc_params,
)
def _lap_w(dinv_hbm, packed_hbm, lw_hbm, dinv_v, pk_v, lw_v):
    w = _wid()
    base = w * ESH
    pltpu.sync_copy(dinv_hbm, dinv_v)
    pltpu.sync_copy(packed_hbm.at[pl.ds(base, ESH)], pk_v)
    sh14 = jnp.full((16,), 14, jnp.int32)
    m14 = jnp.full((16,), 16383, jnp.int32)

    @plsc.parallel_loop(0, ESH, step=16, unroll=10)
    def _(g):
        s = pl.ds(g, 16)
        pk = pk_v[s]
        r = lax.shift_right_logical(pk, sh14)
        c = pk & m14
        lw_v[s] = -(plsc.load_gather(dinv_v, [r]) * plsc.load_gather(dinv_v, [c]))

    pltpu.sync_copy(lw_v, lw_hbm.at[pl.ds(base, ESH)])


def _make_cheb(final_reduce):
    out_type = (jax.ShapeDtypeStruct((NW * 16,), jnp.float32) if final_reduce
                else jax.ShapeDtypeStruct((HP * N,), jnp.float32))
    scratch = ([pltpu.VMEM((N,), jnp.float32)] * (KCH + 3)
               + [pltpu.VMEM((CHUNK,), jnp.int32)] * NBUF
               + [pltpu.VMEM((CHUNK,), jnp.float32)] * NBUF
               + [pltpu.SemaphoreType.DMA] * NBUF)

    @functools.partial(pl.kernel, out_type=out_type, mesh=_mesh,
                       scratch_types=scratch, compiler_params=_sc_params)
    def cheb(zt_hbm, packed_hbm, lw_hbm, out_hbm, z0, z1, z2, z3, z4,
             a_v, b_v, p_v, *ring):
        f = _wid()
        zs = (z0, z1, z2, z3, z4)
        for k in range(KCH):
            pltpu.sync_copy(zt_hbm.at[pl.ds(k * HP * N + f * N, N)], zs[k])

        pk_bufs = ring[0:NBUF]
        lw_bufs = ring[NBUF:2 * NBUF]
        sems = ring[2 * NBUF:3 * NBUF]
        sh14 = jnp.full((16,), 14, jnp.int32)
        m14 = jnp.full((16,), 16383, jnp.int32)
        zero = jnp.zeros((16,), jnp.float32)

        def start(chunk_idx, slot):
            pltpu.async_copy(packed_hbm.at[pl.ds(chunk_idx * CHUNK, CHUNK)],
                             pk_bufs[slot], sems[slot])
            pltpu.async_copy(lw_hbm.at[pl.ds(chunk_idx * CHUNK, CHUNK)],
                             lw_bufs[slot], sems[slot])

        def wait(slot):
            pltpu.make_async_copy(packed_hbm.at[pl.ds(0, CHUNK)],
                                  pk_bufs[slot], sems[slot]).wait()
            pltpu.make_async_copy(lw_hbm.at[pl.ds(0, CHUNK)],
                                  lw_bufs[slot], sems[slot]).wait()

        def prop(src_v):
            for b in range(NBUF - 1):
                start(b, b)

            @plsc.parallel_loop(0, N, step=16, unroll=10)
            def _(i):
                p_v[pl.ds(i, 16)] = zero

            @pl.loop(0, NCHUNK, step=NBUF)
            def _(c):
                for slot in range(NBUF):
                    nxt = c + slot + (NBUF - 1)

                    @pl.when(nxt < NCHUNK)
                    def _():
                        start(nxt, (slot + NBUF - 1) % NBUF)

                    wait(slot)
                    pk_ref = pk_bufs[slot]
                    lw_ref = lw_bufs[slot]

                    @plsc.parallel_loop(0, CHUNK, step=16, unroll=10)
                    def _(g):
                        s = pl.ds(g, 16)
                        pk = pk_ref[s]
                        r = lax.shift_right_logical(pk, sh14)
                        cc = pk & m14
                        vals = plsc.load_gather(src_v, [r]) * lw_ref[s]
                        plsc.addupdate_scatter(p_v, [cc], vals)

        @plsc.parallel_loop(0, N, step=16, unroll=10)
        def _(i):
            s = pl.ds(i, 16)
            a_v[s] = z4[s]

        prop(a_v)

        @plsc.parallel_loop(0, N, step=16, unroll=10)
        def _(i):
            s = pl.ds(i, 16)
            p = p_v[s]
            b_v[s] = z3[s] + p + p

        prop(b_v)

        @plsc.parallel_loop(0, N, step=16, unroll=10)
        def _(i):
            s = pl.ds(i, 16)
            p = p_v[s]
            a_v[s] = z2[s] + p + p - a_v[s]

        prop(a_v)

        @plsc.parallel_loop(0, N, step=16, unroll=10)
        def _(i):
            s = pl.ds(i, 16)
            p = p_v[s]
            b_v[s] = z1[s] + p + p - b_v[s]

        prop(b_v)

        @plsc.parallel_loop(0, N, step=16, unroll=10)
        def _(i):
            s = pl.ds(i, 16)
            a_v[s] = jnp.maximum(z0[s] + p_v[s] - a_v[s], zero)

        if final_reduce:
            def body(i, acc):
                return acc + a_v[pl.ds(i * 16, 16)]

            acc = lax.fori_loop(0, NG, body, zero)
            total = jnp.sum(acc)
            lane = lax.iota(jnp.int32, 16)
            p_v[pl.ds(0, 16)] = jnp.where(lane == 0, total, 0.0)
            pltpu.sync_copy(p_v.at[pl.ds(0, 16)], out_hbm.at[pl.ds(f * 16, 16)])
        else:
            pltpu.sync_copy(a_v, out_hbm.at[pl.ds(f * N, N)])

    return cheb


_cheb_mid = _make_cheb(False)
_cheb_final = _make_cheb(True)


def _tc_prep_body(degp_ref, x_ref, w1t_ref, b1c_ref, dinv_ref, z1t_ref):
    deg = jnp.sum(degp_ref[...], axis=0)
    dinv_ref[...] = jnp.where(deg > 0,
                              lax.rsqrt(jnp.maximum(deg, 1e-12)), 0.0)
    z = lax.dot_general(w1t_ref[...], x_ref[...], (((1,), (1,)), ((), ())),
                        preferred_element_type=jnp.float32)
    z1t_ref[...] = z + b1c_ref[...][:, None]


def _tc_prep(degp, x, w1t, b1c):
    return pl.pallas_call(
        _tc_prep_body,
        out_shape=(jax.ShapeDtypeStruct((N,), jnp.float32),
                   jax.ShapeDtypeStruct((KCH * HP, N), jnp.float32)),
    )(degp, x, w1t, b1c)


def _tc_mid_body(ht_ref, w2t_ref, b2c_ref, z2t_ref):
    z = lax.dot_general(w2t_ref[...], ht_ref[...], (((1,), (0,)), ((), ())),
                        preferred_element_type=jnp.float32)
    z2t_ref[...] = z + b2c_ref[...][:, None]


def _tc_mid(ht, w2t, b2c):
    return pl.pallas_call(
        _tc_mid_body,
        out_shape=jax.ShapeDtypeStruct((KCH * HP, N), jnp.float32),
    )(ht, w2t, b2c)


def _tc_head_body(sums_ref, l1w_ref, l1b_ref, l2w_ref, l2b_ref, out_ref):
    g = jnp.sum(sums_ref[...], axis=1) * (1.0 / N)
    g = g[:H][None, :]
    h = jnp.maximum(
        jnp.dot(g, l1w_ref[...], preferred_element_type=jnp.float32)
        + l1b_ref[...][None, :], 0.0)
    out_ref[...] = (jnp.dot(h, l2w_ref[...], preferred_element_type=jnp.float32)
                    + l2b_ref[...][None, :])


def _tc_head(sums, l1w, l1b, l2w, l2b):
    return pl.pallas_call(
        _tc_head_body,
        out_shape=jax.ShapeDtypeStruct((1, 2), jnp.float32),
    )(sums, l1w, l1b, l2w, l2b)


def kernel(x, edge_index, W1, b1, W2, b2, lin1_w, lin1_b, lin2_w, lin2_b):
    w1t = jnp.pad(W1, ((0, 0), (0, 0), (0, HP - H))) \
        .transpose(0, 2, 1).reshape(KCH * HP, DF)
    b1c = jnp.concatenate([jnp.pad(b1, (0, HP - H)),
                           jnp.zeros(((KCH - 1) * HP,), jnp.float32)])
    w2t = jnp.pad(W2, ((0, 0), (0, HP - H), (0, HP - H))) \
        .transpose(0, 2, 1).reshape(KCH * HP, HP)
    b2c = jnp.concatenate([jnp.pad(b2, (0, HP - H)),
                           jnp.zeros(((KCH - 1) * HP,), jnp.float32)])

    degp, packed = _deg_pack(edge_index.reshape(-1))
    dinv, z1t = _tc_prep(degp.reshape(NW, N), x, w1t, b1c)
    lw = _lap_w(dinv, packed)
    ht = _cheb_mid(z1t.reshape(-1), packed, lw)
    z2t = _tc_mid(ht.reshape(HP, N), w2t, b2c)
    sums = _cheb_final(z2t.reshape(-1), packed, lw)
    return _tc_head(sums.reshape(NW, 16), lin1_w, lin1_b, lin2_w, lin2_b)


# speedup vs baseline: 1.2267x; 1.2267x over previous
import functools

import jax
import jax.numpy as jnp
from jax import lax
from jax.experimental import pallas as pl
from jax.experimental.pallas import tpu as pltpu, tpu_sc as plsc

N = 10000
E = 320000
DF = 128
H = 30
HP = 32
KCH = 5
NW = 32
ESH = E // NW
CHUNK = 4000
NBUF = 4
NCHUNK = E // CHUNK
GROUPS = CHUNK // 16
NG = N // 16

_mesh = plsc.VectorSubcoreMesh(core_axis_name="c", subcore_axis_name="s")
_sc_params = pltpu.CompilerParams(needs_layout_passes=False)


def _wid():
    return lax.axis_index("s") * 2 + lax.axis_index("c")


@functools.partial(
    pl.kernel,
    out_type=(jax.ShapeDtypeStruct((NW * N,), jnp.float32),
              jax.ShapeDtypeStruct((E,), jnp.int32)),
    mesh=_mesh,
    scratch_types=[pltpu.VMEM((ESH,), jnp.int32),
                   pltpu.VMEM((ESH,), jnp.int32),
                   pltpu.VMEM((ESH,), jnp.int32),
                   pltpu.VMEM((N,), jnp.float32)],
    compiler_params=_sc_params,
)
def _deg_pack(edge_hbm, degp_hbm, packed_hbm, row_v, col_v, pk_v, deg_v):
    w = _wid()
    base = w * ESH
    pltpu.sync_copy(edge_hbm.at[pl.ds(base, ESH)], row_v)
    pltpu.sync_copy(edge_hbm.at[pl.ds(E + base, ESH)], col_v)
    zero = jnp.zeros((16,), jnp.float32)

    @plsc.parallel_loop(0, N, step=16, unroll=10)
    def _(i):
        deg_v[pl.ds(i, 16)] = zero

    ones = jnp.ones((16,), jnp.float32)
    k14 = jnp.full((16,), 16384, jnp.int32)

    @plsc.parallel_loop(0, ESH, step=16, unroll=10)
    def _(g):
        s = pl.ds(g, 16)
        r = row_v[s]
        pk_v[s] = r * k14 + col_v[s]
        plsc.addupdate_scatter(deg_v, [r], ones)

    pltpu.sync_copy(pk_v, packed_hbm.at[pl.ds(base, ESH)])
    pltpu.sync_copy(deg_v, degp_hbm.at[pl.ds(w * N, N)])


def _make_cheb(final_reduce):
    out_type = (jax.ShapeDtypeStruct((NW * 16,), jnp.float32) if final_reduce
                else jax.ShapeDtypeStruct((HP * N,), jnp.float32))
    scratch = ([pltpu.VMEM((N,), jnp.float32)] * (KCH + 5)
               + [pltpu.VMEM((CHUNK,), jnp.int32)] * NBUF
               + [pltpu.SemaphoreType.DMA] * NBUF)

    @functools.partial(pl.kernel, out_type=out_type, mesh=_mesh,
                       scratch_types=scratch, compiler_params=_sc_params)
    def cheb(zt_hbm, packed_hbm, dinv_hbm, out_hbm, z0, z1, z2, z3, z4,
             a_v, b_v, p_v, s_v, dinv_v, *ring):
        f = _wid()
        zs = (z0, z1, z2, z3, z4)
        pltpu.sync_copy(dinv_hbm, dinv_v)
        for k in range(KCH):
            pltpu.sync_copy(zt_hbm.at[pl.ds(k * HP * N + f * N, N)], zs[k])

        pk_bufs = ring[0:NBUF]
        sems = ring[NBUF:2 * NBUF]
        sh14 = jnp.full((16,), 14, jnp.int32)
        m14 = jnp.full((16,), 16383, jnp.int32)
        zero = jnp.zeros((16,), jnp.float32)

        def start(chunk_idx, slot):
            pltpu.async_copy(packed_hbm.at[pl.ds(chunk_idx * CHUNK, CHUNK)],
                             pk_bufs[slot], sems[slot])

        def wait(slot):
            pltpu.make_async_copy(packed_hbm.at[pl.ds(0, CHUNK)],
                                  pk_bufs[slot], sems[slot]).wait()

        def prop():
            for b in range(NBUF - 1):
                start(b, b)

            @plsc.parallel_loop(0, N, step=16, unroll=10)
            def _(i):
                p_v[pl.ds(i, 16)] = zero

            @pl.loop(0, NCHUNK, step=NBUF)
            def _(c):
                for slot in range(NBUF):
                    nxt = c + slot + (NBUF - 1)

                    @pl.when(nxt < NCHUNK)
                    def _():
                        start(nxt, (slot + NBUF - 1) % NBUF)

                    wait(slot)
                    pk_ref = pk_bufs[slot]

                    @plsc.parallel_loop(0, CHUNK, step=16, unroll=10)
                    def _(g):
                        s = pl.ds(g, 16)
                        pk = pk_ref[s]
                        r = lax.shift_right_logical(pk, sh14)
                        cc = pk & m14
                        plsc.addupdate_scatter(p_v, [cc],
                                               plsc.load_gather(s_v, [r]))

        @plsc.parallel_loop(0, N, step=16, unroll=10)
        def _(i):
            s = pl.ds(i, 16)
            a = z4[s]
            a_v[s] = a
            s_v[s] = dinv_v[s] * a

        prop()

        @plsc.parallel_loop(0, N, step=16, unroll=10)
        def _(i):
            s = pl.ds(i, 16)
            pd = dinv_v[s]
            q = pd * p_v[s]
            b = z3[s] - q - q
            b_v[s] = b
            s_v[s] = pd * b

        prop()

        @plsc.parallel_loop(0, N, step=16, unroll=10)
        def _(i):
            s = pl.ds(i, 16)
            pd = dinv_v[s]
            q = pd * p_v[s]
            a = z2[s] - q - q - a_v[s]
            a_v[s] = a
            s_v[s] = pd * a

        prop()

        @plsc.parallel_loop(0, N, step=16, unroll=10)
        def _(i):
            s = pl.ds(i, 16)
            pd = dinv_v[s]
            q = pd * p_v[s]
            b = z1[s] - q - q - b_v[s]
            b_v[s] = b
            s_v[s] = pd * b

        prop()

        @plsc.parallel_loop(0, N, step=16, unroll=10)
        def _(i):
            s = pl.ds(i, 16)
            q = dinv_v[s] * p_v[s]
            a_v[s] = jnp.maximum(z0[s] - q - a_v[s], zero)

        if final_reduce:
            def body(i, acc):
                return acc + a_v[pl.ds(i * 16, 16)]

            acc = lax.fori_loop(0, NG, body, zero)
            total = jnp.sum(acc)
            lane = lax.iota(jnp.int32, 16)
            p_v[pl.ds(0, 16)] = jnp.where(lane == 0, total, 0.0)
            pltpu.sync_copy(p_v.at[pl.ds(0, 16)], out_hbm.at[pl.ds(f * 16, 16)])
        else:
            pltpu.sync_copy(a_v, out_hbm.at[pl.ds(f * N, N)])

    return cheb


_cheb_mid = _make_cheb(False)
_cheb_final = _make_cheb(True)


def _tc_prep_body(degp_ref, x_ref, w1t_ref, b1c_ref, dinv_ref, z1t_ref):
    deg = jnp.sum(degp_ref[...], axis=0)
    dinv_ref[...] = jnp.where(deg > 0,
                              lax.rsqrt(jnp.maximum(deg, 1e-12)), 0.0)
    z = lax.dot_general(w1t_ref[...], x_ref[...], (((1,), (1,)), ((), ())),
                        preferred_element_type=jnp.float32)
    z1t_ref[...] = z + b1c_ref[...][:, None]


def _tc_prep(degp, x, w1t, b1c):
    return pl.pallas_call(
        _tc_prep_body,
        out_shape=(jax.ShapeDtypeStruct((N,), jnp.float32),
                   jax.ShapeDtypeStruct((KCH * HP, N), jnp.float32)),
    )(degp, x, w1t, b1c)


def _tc_mid_body(ht_ref, w2t_ref, b2c_ref, z2t_ref):
    z = lax.dot_general(w2t_ref[...], ht_ref[...], (((1,), (0,)), ((), ())),
                        preferred_element_type=jnp.float32)
    z2t_ref[...] = z + b2c_ref[...][:, None]


def _tc_mid(ht, w2t, b2c):
    return pl.pallas_call(
        _tc_mid_body,
        out_shape=jax.ShapeDtypeStruct((KCH * HP, N), jnp.float32),
    )(ht, w2t, b2c)


def _tc_head_body(sums_ref, l1w_ref, l1b_ref, l2w_ref, l2b_ref, out_ref):
    g = jnp.sum(sums_ref[...], axis=1) * (1.0 / N)
    g = g[:H][None, :]
    h = jnp.maximum(
        jnp.dot(g, l1w_ref[...], preferred_element_type=jnp.float32)
        + l1b_ref[...][None, :], 0.0)
    out_ref[...] = (jnp.dot(h, l2w_ref[...], preferred_element_type=jnp.float32)
                    + l2b_ref[...][None, :])


def _tc_head(sums, l1w, l1b, l2w, l2b):
    return pl.pallas_call(
        _tc_head_body,
        out_shape=jax.ShapeDtypeStruct((1, 2), jnp.float32),
    )(sums, l1w, l1b, l2w, l2b)


def kernel(x, edge_index, W1, b1, W2, b2, lin1_w, lin1_b, lin2_w, lin2_b):
    w1t = jnp.pad(W1, ((0, 0), (0, 0), (0, HP - H))) \
        .transpose(0, 2, 1).reshape(KCH * HP, DF)
    b1c = jnp.concatenate([jnp.pad(b1, (0, HP - H)),
                           jnp.zeros(((KCH - 1) * HP,), jnp.float32)])
    w2t = jnp.pad(W2, ((0, 0), (0, HP - H), (0, HP - H))) \
        .transpose(0, 2, 1).reshape(KCH * HP, HP)
    b2c = jnp.concatenate([jnp.pad(b2, (0, HP - H)),
                           jnp.zeros(((KCH - 1) * HP,), jnp.float32)])

    degp, packed = _deg_pack(edge_index.reshape(-1))
    dinv, z1t = _tc_prep(degp.reshape(NW, N), x, w1t, b1c)
    ht = _cheb_mid(z1t.reshape(-1), packed, dinv)
    z2t = _tc_mid(ht.reshape(HP, N), w2t, b2c)
    sums = _cheb_final(z2t.reshape(-1), packed, dinv)
    return _tc_head(sums.reshape(NW, 16), lin1_w, lin1_b, lin2_w, lin2_b)
